# Initial kernel scaffold; baseline (speedup 1.0000x reference)
import functools

import jax
import jax.numpy as jnp
import numpy as np
from jax import lax
from jax.experimental import pallas as pl
from jax.experimental.pallas import tpu as pltpu
from jax.experimental.pallas import tpu_sc as plsc

_POOLING_COUNTS = [15, 8, 8]
_N = 64

_WBASE = {1: 0, 2: 80, 4: 160, 8: 240, 16: 320, 32: 400}
_T_SIZE = 512
_ZERO_IDX = 511

_NUM_CORES = 2
_NUM_SUBCORES = 16
_NW = _NUM_CORES * _NUM_SUBCORES
_B, _D = 32, 512
_ROWS = _B * _D
_RPW = _ROWS // _NW
_CHUNK = 8
_NCHUNK = _RPW // _CHUNK
_ROW_WORDS = _N * _N
_OUT_WORDS = _CHUNK * _ROW_WORDS
_TOTAL_WORDS = _ROWS * _ROW_WORDS


def _cell_specs():
    maskij = []
    stride, offset = 1, 0
    for c in _POOLING_COUNTS:
        for _ in range(c):
            offset += stride
            i = np.arange(0, _N - offset, stride)
            j = np.arange(offset, _N, stride)
            maskij.append((i, j))
        stride *= 2
    return maskij


def _build_tables():
    cells = []
    for i in range(_N):
        cells.append((i * _N + i, _WBASE[1] + i, _WBASE[1] + i,
                      _ZERO_IDX, _ZERO_IDX))
    for (ii, jj) in _cell_specs():
        for i, j in zip(ii.tolist(), jj.tolist()):
            length = j - i + 1
            w = 1
            while w * 2 <= length:
                w *= 2
            w = min(w, 32)
            cells.append((i * _N + j, _WBASE[w] + i,
                          _WBASE[w] + (j - w + 1), i, j))
    arr = np.array(cells, dtype=np.int32)
    assert arr.shape[0] % 16 == 0
    return np.concatenate([arr[:, 1], arr[:, 2], arr[:, 3], arr[:, 4],
                           arr[:, 0]]), arr.shape[0]


_TAB_NP, _NCELLS = _build_tables()
_NGROUPS = _NCELLS // 16
_OFF_A = 0
_OFF_B = _NCELLS
_OFF_XI = 2 * _NCELLS
_OFF_XJ = 3 * _NCELLS
_OFF_P = 4 * _NCELLS

_mesh = plsc.VectorSubcoreMesh(
    core_axis_name="c", subcore_axis_name="s",
    num_cores=_NUM_CORES, num_subcores=_NUM_SUBCORES)


@functools.partial(
    pl.kernel,
    out_type=jax.ShapeDtypeStruct((_TOTAL_WORDS,), jnp.float32),
    mesh=_mesh,
    scratch_types=[
        pltpu.VMEM((_T_SIZE,), jnp.float32),
        pltpu.VMEM((_CHUNK * _N,), jnp.float32),
        pltpu.VMEM((_OUT_WORDS,), jnp.float32),
        pltpu.VMEM((5 * _NCELLS,), jnp.int32),
    ],
)
def _sc_kernel(x_hbm, tab_hbm, out_hbm, t_v, in_v, out_v, tab_v):
    wid = lax.axis_index("s") * _NUM_CORES + lax.axis_index("c")
    pltpu.sync_copy(tab_hbm, tab_v)

    z = jnp.zeros((16,), jnp.float32)
    for k in range(_T_SIZE // 16):
        t_v[pl.ds(k * 16, 16)] = z

    def zero_body(i, carry):
        out_v[pl.ds(i * 16, 16)] = z
        return carry

    lax.fori_loop(0, _OUT_WORDS // 16, zero_body, 0)

    def chunk_body(ci, carry):
        row0 = wid * _RPW + ci * _CHUNK
        pltpu.sync_copy(x_hbm.at[pl.ds(row0 * _N, _CHUNK * _N)], in_v)

        def row_body(r, rcarry):
            for k in range(_N // 16):
                t_v[pl.ds(k * 16, 16)] = in_v[pl.ds(r * _N + k * 16, 16)]
            for w, shift in ((2, 1), (4, 2), (8, 4), (16, 8), (32, 16)):
                src = _WBASE[w // 2]
                dst = _WBASE[w]
                for k in range(_N // 16):
                    lo = k * 16
                    t_v[pl.ds(dst + lo, 16)] = jnp.maximum(
                        t_v[pl.ds(src + lo, 16)],
                        t_v[pl.ds(src + lo + shift, 16)])
            rbase = r * _ROW_WORDS
            for g in range(_NGROUPS):
                o = g * 16
                ia = tab_v[pl.ds(_OFF_A + o, 16)]
                ib = tab_v[pl.ds(_OFF_B + o, 16)]
                ixi = tab_v[pl.ds(_OFF_XI + o, 16)]
                ixj = tab_v[pl.ds(_OFF_XJ + o, 16)]
                ip = tab_v[pl.ds(_OFF_P + o, 16)]
                va = plsc.load_gather(t_v, [ia])
                vb = plsc.load_gather(t_v, [ib])
                vxi = plsc.load_gather(t_v, [ixi])
                vxj = plsc.load_gather(t_v, [ixj])
                val = jnp.maximum(va, vb) + vxi + vxj
                plsc.store_scatter(out_v, [ip + rbase], val)
            return rcarry

        lax.fori_loop(0, _CHUNK, row_body, 0)
        pltpu.sync_copy(
            out_v, out_hbm.at[pl.ds(row0 * _ROW_WORDS, _OUT_WORDS)])
        return carry

    lax.fori_loop(0, _NCHUNK, chunk_body, 0)


def kernel(x):
    B, D, n = x.shape
    tab = jnp.asarray(_TAB_NP)
    out = _sc_kernel(x.reshape(-1), tab)
    return out.reshape(B, D, n, n)


# speedup vs baseline: 1.0328x; 1.0328x over previous
import functools

import jax
import jax.numpy as jnp
import numpy as np
from jax import lax
from jax.experimental import pallas as pl
from jax.experimental.pallas import tpu as pltpu
from jax.experimental.pallas import tpu_sc as plsc

_POOLING_COUNTS = [15, 8, 8]
_N = 64

_WBASE = {1: 0, 2: 80, 4: 160, 8: 240, 16: 320, 32: 400}
_T_SIZE = 512
_ZERO_IDX = 511

_NUM_CORES = 2
_NUM_SUBCORES = 16
_NW = _NUM_CORES * _NUM_SUBCORES
_B, _D = 32, 512
_ROWS = _B * _D
_RPW = _ROWS // _NW
_CHUNK = 8
_NCHUNK = _RPW // _CHUNK
_ROW_WORDS = _N * _N
_OUT_WORDS = _CHUNK * _ROW_WORDS
_TOTAL_WORDS = _ROWS * _ROW_WORDS


def _cell_specs():
    maskij = []
    stride, offset = 1, 0
    for c in _POOLING_COUNTS:
        for _ in range(c):
            offset += stride
            i = np.arange(0, _N - offset, stride)
            j = np.arange(offset, _N, stride)
            maskij.append((i, j))
        stride *= 2
    return maskij


def _build_tables():
    cells = []
    for i in range(_N):
        cells.append((i * _N + i, _WBASE[1] + i, _WBASE[1] + i,
                      _ZERO_IDX, _ZERO_IDX))
    for (ii, jj) in _cell_specs():
        for i, j in zip(ii.tolist(), jj.tolist()):
            length = j - i + 1
            w = 1
            while w * 2 <= length:
                w *= 2
            w = min(w, 32)
            cells.append((i * _N + j, _WBASE[w] + i,
                          _WBASE[w] + (j - w + 1), i, j))
    arr = np.array(cells, dtype=np.int32)
    assert arr.shape[0] % 16 == 0
    return np.concatenate([arr[:, 1], arr[:, 2], arr[:, 3], arr[:, 4],
                           arr[:, 0]]), arr.shape[0]


_TAB_NP, _NCELLS = _build_tables()
_NGROUPS = _NCELLS // 16
_OFF_A = 0
_OFF_B = _NCELLS
_OFF_XI = 2 * _NCELLS
_OFF_XJ = 3 * _NCELLS
_OFF_P = 4 * _NCELLS

_mesh = plsc.VectorSubcoreMesh(
    core_axis_name="c", subcore_axis_name="s",
    num_cores=_NUM_CORES, num_subcores=_NUM_SUBCORES)


@functools.partial(
    pl.kernel,
    out_type=jax.ShapeDtypeStruct((_TOTAL_WORDS,), jnp.float32),
    mesh=_mesh,
    compiler_params=pltpu.CompilerParams(needs_layout_passes=False),
    scratch_types=[
        pltpu.VMEM((_T_SIZE,), jnp.float32),
        pltpu.VMEM((_CHUNK * _N,), jnp.float32),
        pltpu.VMEM((_OUT_WORDS,), jnp.float32),
        pltpu.VMEM((5 * _NCELLS,), jnp.int32),
    ],
)
def _sc_kernel(x_hbm, tab_hbm, out_hbm, t_v, in_v, out_v, tab_v):
    wid = lax.axis_index("s") * _NUM_CORES + lax.axis_index("c")
    pltpu.sync_copy(tab_hbm, tab_v)

    z = jnp.zeros((16,), jnp.float32)
    for k in range(_T_SIZE // 16):
        t_v[pl.ds(k * 16, 16)] = z

    def zero_body(i, carry):
        out_v[pl.ds(i * 16, 16)] = z
        return carry

    lax.fori_loop(0, _OUT_WORDS // 16, zero_body, 0)

    def chunk_body(ci, carry):
        row0 = wid * _RPW + ci * _CHUNK
        pltpu.sync_copy(x_hbm.at[pl.ds(row0 * _N, _CHUNK * _N)], in_v)

        def row_body(r, rcarry):
            for k in range(_N // 16):
                t_v[pl.ds(k * 16, 16)] = in_v[pl.ds(r * _N + k * 16, 16)]
            for w, shift in ((2, 1), (4, 2), (8, 4), (16, 8), (32, 16)):
                src = _WBASE[w // 2]
                dst = _WBASE[w]
                for k in range(_N // 16):
                    lo = k * 16
                    t_v[pl.ds(dst + lo, 16)] = jnp.maximum(
                        t_v[pl.ds(src + lo, 16)],
                        t_v[pl.ds(src + lo + shift, 16)])
            rbase = r * _ROW_WORDS
            for g in range(_NGROUPS):
                o = g * 16
                ia = tab_v[pl.ds(_OFF_A + o, 16)]
                ib = tab_v[pl.ds(_OFF_B + o, 16)]
                ixi = tab_v[pl.ds(_OFF_XI + o, 16)]
                ixj = tab_v[pl.ds(_OFF_XJ + o, 16)]
                ip = tab_v[pl.ds(_OFF_P + o, 16)]
                va = plsc.load_gather(t_v, [ia])
                vb = plsc.load_gather(t_v, [ib])
                vxi = plsc.load_gather(t_v, [ixi])
                vxj = plsc.load_gather(t_v, [ixj])
                val = jnp.maximum(va, vb) + vxi + vxj
                plsc.store_scatter(out_v, [ip + rbase], val)
            return rcarry

        lax.fori_loop(0, _CHUNK, row_body, 0)
        pltpu.sync_copy(
            out_v, out_hbm.at[pl.ds(row0 * _ROW_WORDS, _OUT_WORDS)])
        return carry

    lax.fori_loop(0, _NCHUNK, chunk_body, 0)


def kernel(x):
    B, D, n = x.shape
    tab = jnp.asarray(_TAB_NP)
    out = _sc_kernel(x.reshape(-1), tab)
    return out.reshape(B, D, n, n)


# speedup vs baseline: 2.5213x; 2.4413x over previous
import functools

import jax
import jax.numpy as jnp
import numpy as np
from jax import lax
from jax.experimental import pallas as pl
from jax.experimental.pallas import tpu as pltpu
from jax.experimental.pallas import tpu_sc as plsc

_POOLING_COUNTS = [15, 8, 8]
_N = 64
_B, _D = 32, 512

_NUM_CORES = 2
_NUM_SUBCORES = 16

_DB = 4
_DBW = 128
_IB = 16
_IBW = 4

_XROW = 0
_P2 = 64
_P4 = 127
_P8 = 188
_P16 = 245
_P32 = 294
_ZROW = 327
_T_ROWS = 328
_PYR = (
    (_P2, _XROW, 1, 63),
    (_P4, _P2, 2, 61),
    (_P8, _P4, 4, 57),
    (_P16, _P8, 8, 49),
    (_P32, _P16, 16, 33),
)
_WROW = {1: _XROW, 2: _P2, 4: _P4, 8: _P8, 16: _P16, 32: _P32}


def _cell_specs():
    maskij = []
    stride, offset = 1, 0
    for c in _POOLING_COUNTS:
        for _ in range(c):
            offset += stride
            i = np.arange(0, _N - offset, stride)
            j = np.arange(offset, _N, stride)
            maskij.append((i, j))
        stride *= 2
    return maskij


def _build_tables():
    cells = []
    for i in range(_N):
        cells.append((i, i, _XROW + i, _XROW + i, _ZROW, _ZROW))
    for (ii, jj) in _cell_specs():
        for i, j in zip(ii.tolist(), jj.tolist()):
            length = j - i + 1
            w = 1
            while w * 2 <= length:
                w *= 2
            w = min(w, 32)
            cells.append((i, j, _WROW[w] + i, _WROW[w] + (j - w + 1),
                          _XROW + i, _XROW + j))
    groups = [[] for _ in range(_IB)]
    for c in cells:
        groups[c[0] // _IBW].append(c)
    rows = []
    cum = [0]
    for ib, g in enumerate(groups):
        if len(g) % 2:
            g = g + [(ib * _IBW + 3, 0, _ZROW, _ZROW, _ZROW, _ZROW)]
        for (i, j, a, bb, xi, xj) in g:
            rows.append((i % _IBW, j, a * _DBW, bb * _DBW,
                         xi * _DBW, xj * _DBW, 0, 0))
        cum.append(cum[-1] + len(g))
    arr = np.array(rows, dtype=np.int32)
    return arr.reshape(-1), arr.shape[0], cum


_TAB_NP, _NCELLS, _CUM = _build_tables()

_mesh = plsc.VectorSubcoreMesh(
    core_axis_name="c", subcore_axis_name="s",
    num_cores=_NUM_CORES, num_subcores=_NUM_SUBCORES)


@functools.partial(
    pl.kernel,
    out_type=jax.ShapeDtypeStruct((_B, _N, _N, _D), jnp.float32),
    mesh=_mesh,
    compiler_params=pltpu.CompilerParams(
        needs_layout_passes=False, use_tc_tiling_on_sc=True),
    scratch_types=[
        pltpu.VMEM((_T_ROWS * _DBW,), jnp.float32),
        pltpu.VMEM((_DBW * _N,), jnp.float32),
        pltpu.VMEM((_IBW, _N, _DBW), jnp.float32),
        pltpu.VMEM((8 * _NCELLS,), jnp.int32),
    ],
)
def _sc_kernel(x_hbm, tab_hbm, out_hbm, t_v, stage_v, out_v, tab_v):
    b = lax.axis_index("s") * _NUM_CORES + lax.axis_index("c")
    pltpu.sync_copy(tab_hbm, tab_v)

    z = jnp.zeros((16,), jnp.float32)
    for dd in range(8):
        t_v[pl.ds(_ZROW * _DBW + dd * 16, 16)] = z

    def zero_all(m, carry):
        ir = lax.shift_right_logical(m, 9)
        j = lax.bitwise_and(lax.shift_right_logical(m, 3), 63)
        dd16 = lax.bitwise_and(m, 7) * 16
        out_v[ir, j, pl.ds(dd16, 16)] = z
        return carry

    lax.fori_loop(0, _IBW * _N * _DBW // 16, zero_all, 0)

    iota64 = lax.iota(jnp.int32, 16) * 64

    def db_body(db, carry):
        pltpu.sync_copy(
            x_hbm.at[pl.ds(b * (_D * _N) + db * (_DBW * _N), _DBW * _N)],
            stage_v)

        def tr_body(dd, c2):
            base = dd * 1024
            for i in range(_N):
                v = plsc.load_gather(stage_v, [iota64 + (base + i)])
                t_v[pl.ds(i * _DBW + dd * 16, 16)] = v
            return c2

        lax.fori_loop(0, 8, tr_body, 0)

        def pyr_body(dd, c2):
            o = dd * 16
            for (dst, src, shift, cnt) in _PYR:
                for k in range(cnt):
                    va = t_v[pl.ds((src + k) * _DBW + o, 16)]
                    vb = t_v[pl.ds((src + k + shift) * _DBW + o, 16)]
                    t_v[pl.ds((dst + k) * _DBW + o, 16)] = jnp.maximum(va, vb)
            return c2

        lax.fori_loop(0, 8, pyr_body, 0)

        for ib in range(_IB):
            def cell_body(p, c2):
                meta = tab_v[pl.ds(p * 16, 16)]
                for h in range(2):
                    ir = meta[8 * h + 0]
                    j = meta[8 * h + 1]
                    a = meta[8 * h + 2]
                    bo = meta[8 * h + 3]
                    xi = meta[8 * h + 4]
                    xj = meta[8 * h + 5]
                    for dd in range(8):
                        o = dd * 16
                        va = t_v[pl.ds(a + o, 16)]
                        vb = t_v[pl.ds(bo + o, 16)]
                        vxi = t_v[pl.ds(xi + o, 16)]
                        vxj = t_v[pl.ds(xj + o, 16)]
                        out_v[ir, j, pl.ds(o, 16)] = (
                            jnp.maximum(va, vb) + vxi + vxj)
                return c2

            lax.fori_loop(_CUM[ib] // 2, _CUM[ib + 1] // 2, cell_body, 0)
            pltpu.sync_copy(
                out_v,
                out_hbm.at[b, pl.ds(ib * _IBW, _IBW), :,
                           pl.ds(db * _DBW, _DBW)])

            def rezero_body(p, c2):
                meta = tab_v[pl.ds(p * 16, 16)]
                for h in range(2):
                    ir = meta[8 * h + 0]
                    j = meta[8 * h + 1]
                    for dd in range(8):
                        out_v[ir, j, pl.ds(dd * 16, 16)] = z
                return c2

            lax.fori_loop(_CUM[ib] // 2, _CUM[ib + 1] // 2, rezero_body, 0)
        return carry

    lax.fori_loop(0, _DB, db_body, 0)


def kernel(x):
    B, D, n = x.shape
    tab = jnp.asarray(_TAB_NP)
    out_t = _sc_kernel(x.reshape(-1), tab)
    return jnp.transpose(out_t, (0, 3, 1, 2))


# speedup vs baseline: 2.9951x; 1.1879x over previous
import functools

import jax
import jax.numpy as jnp
import numpy as np
from jax import lax
from jax.experimental import pallas as pl
from jax.experimental.pallas import tpu as pltpu
from jax.experimental.pallas import tpu_sc as plsc

_POOLING_COUNTS = [15, 8, 8]
_N = 64
_B, _D = 32, 512

_NUM_CORES = 2
_NUM_SUBCORES = 16

_DB = 4
_DBW = 128
_IB = 16
_IBW = 4

_XROW = 0
_P2 = 64
_P4 = 127
_P8 = 188
_P16 = 245
_P32 = 294
_ZROW = 327
_T_ROWS = 328
_PYR = (
    (_P2, _XROW, 1, 63),
    (_P4, _P2, 2, 61),
    (_P8, _P4, 4, 57),
    (_P16, _P8, 8, 49),
    (_P32, _P16, 16, 33),
)
_WROW = {1: _XROW, 2: _P2, 4: _P4, 8: _P8, 16: _P16, 32: _P32}


def _cell_specs():
    maskij = []
    stride, offset = 1, 0
    for c in _POOLING_COUNTS:
        for _ in range(c):
            offset += stride
            i = np.arange(0, _N - offset, stride)
            j = np.arange(offset, _N, stride)
            maskij.append((i, j))
        stride *= 2
    return maskij


def _build_tables():
    cells = []
    for i in range(_N):
        cells.append((i, i, _XROW + i, _XROW + i, _ZROW, _ZROW))
    for (ii, jj) in _cell_specs():
        for i, j in zip(ii.tolist(), jj.tolist()):
            length = j - i + 1
            w = 1
            while w * 2 <= length:
                w *= 2
            w = min(w, 32)
            cells.append((i, j, _WROW[w] + i, _WROW[w] + (j - w + 1),
                          _XROW + i, _XROW + j))
    groups = [[] for _ in range(_IB)]
    for c in cells:
        groups[c[0] // _IBW].append(c)
    rows = []
    cum = [0]
    for ib, g in enumerate(groups):
        if len(g) % 2:
            g = g + [(ib * _IBW + 3, 0, _ZROW, _ZROW, _ZROW, _ZROW)]
        for (i, j, a, bb, xi, xj) in g:
            rows.append((i % _IBW, j, a * _DBW, bb * _DBW,
                         xi * _DBW, xj * _DBW, 0, 0))
        cum.append(cum[-1] + len(g))
    arr = np.array(rows, dtype=np.int32)
    return arr.reshape(-1), arr.shape[0], cum


_TAB_NP, _NCELLS, _CUM = _build_tables()

_mesh = plsc.VectorSubcoreMesh(
    core_axis_name="c", subcore_axis_name="s",
    num_cores=_NUM_CORES, num_subcores=_NUM_SUBCORES)


@functools.partial(
    pl.kernel,
    out_type=jax.ShapeDtypeStruct((_B, _N, _N, _D), jnp.float32),
    mesh=_mesh,
    compiler_params=pltpu.CompilerParams(
        needs_layout_passes=False, use_tc_tiling_on_sc=True),
    scratch_types=[
        pltpu.VMEM((_T_ROWS * _DBW,), jnp.float32),
        pltpu.VMEM((_DBW * _N,), jnp.float32),
        pltpu.VMEM((_IBW, _N, _DBW), jnp.float32),
        pltpu.VMEM((_IBW, _N, _DBW), jnp.float32),
        pltpu.VMEM((8 * _NCELLS,), jnp.int32),
        pltpu.SemaphoreType.DMA,
        pltpu.SemaphoreType.DMA,
    ],
)
def _sc_kernel(x_hbm, tab_hbm, out_hbm, t_v, stage_v, out_v0, out_v1,
               tab_v, sem0, sem1):
    b = lax.axis_index("s") * _NUM_CORES + lax.axis_index("c")
    pltpu.sync_copy(tab_hbm, tab_v)

    z = jnp.zeros((16,), jnp.float32)
    for dd in range(8):
        t_v[pl.ds(_ZROW * _DBW + dd * 16, 16)] = z

    def zero_all(m, carry):
        ir = lax.shift_right_logical(m, 9)
        j = lax.bitwise_and(lax.shift_right_logical(m, 3), 63)
        dd16 = lax.bitwise_and(m, 7) * 16
        out_v0[ir, j, pl.ds(dd16, 16)] = z
        out_v1[ir, j, pl.ds(dd16, 16)] = z
        return carry

    lax.fori_loop(0, _IBW * _N * _DBW // 16, zero_all, 0)

    iota64 = lax.iota(jnp.int32, 16) * 64

    def db_body(db, carry):
        pltpu.sync_copy(
            x_hbm.at[pl.ds(b * (_D * _N) + db * (_DBW * _N), _DBW * _N)],
            stage_v)

        def tr_body(dd, c2):
            base = dd * 1024
            for i in range(_N):
                v = plsc.load_gather(stage_v, [iota64 + (base + i)])
                t_v[pl.ds(i * _DBW + dd * 16, 16)] = v
            return c2

        lax.fori_loop(0, 8, tr_body, 0)

        def pyr_body(dd, c2):
            o = dd * 16
            for (dst, src, shift, cnt) in _PYR:
                for k in range(cnt):
                    va = t_v[pl.ds((src + k) * _DBW + o, 16)]
                    vb = t_v[pl.ds((src + k + shift) * _DBW + o, 16)]
                    t_v[pl.ds((dst + k) * _DBW + o, 16)] = jnp.maximum(va, vb)
            return c2

        lax.fori_loop(0, 8, pyr_body, 0)

        for ib in range(_IB):
            buf = out_v0 if ib % 2 == 0 else out_v1
            sem = sem0 if ib % 2 == 0 else sem1
            dst = out_hbm.at[b, pl.ds(ib * _IBW, _IBW), :,
                             pl.ds(db * _DBW, _DBW)]

            def rezero_loop(lo, hi):
                def rezero_body(p, c2):
                    meta = tab_v[pl.ds(p * 16, 16)]
                    for h in range(2):
                        ir = meta[8 * h + 0]
                        j = meta[8 * h + 1]
                        for dd in range(8):
                            buf[ir, j, pl.ds(dd * 16, 16)] = z
                    return c2
                lax.fori_loop(lo // 2, hi // 2, rezero_body, 0)

            if ib >= 2:
                pltpu.make_async_copy(buf, dst, sem).wait()
                rezero_loop(_CUM[ib - 2], _CUM[ib - 1])
            else:
                @pl.when(db > 0)
                def _():
                    pltpu.make_async_copy(buf, dst, sem).wait()
                    rezero_loop(_CUM[ib + 14], _CUM[ib + 15])

            def cell_body(p, c2):
                meta = tab_v[pl.ds(p * 16, 16)]
                for h in range(2):
                    ir = meta[8 * h + 0]
                    j = meta[8 * h + 1]
                    a = meta[8 * h + 2]
                    bo = meta[8 * h + 3]
                    xi = meta[8 * h + 4]
                    xj = meta[8 * h + 5]
                    for dd in range(8):
                        o = dd * 16
                        va = t_v[pl.ds(a + o, 16)]
                        vb = t_v[pl.ds(bo + o, 16)]
                        vxi = t_v[pl.ds(xi + o, 16)]
                        vxj = t_v[pl.ds(xj + o, 16)]
                        buf[ir, j, pl.ds(o, 16)] = (
                            jnp.maximum(va, vb) + vxi + vxj)
                return c2

            lax.fori_loop(_CUM[ib] // 2, _CUM[ib + 1] // 2, cell_body, 0)
            pltpu.async_copy(buf, dst, sem)
        return carry

    lax.fori_loop(0, _DB, db_body, 0)

    for (buf, sem, ib) in ((out_v0, sem0, 14), (out_v1, sem1, 15)):
        pltpu.make_async_copy(
            buf,
            out_hbm.at[b, pl.ds(ib * _IBW, _IBW), :,
                       pl.ds((_DB - 1) * _DBW, _DBW)],
            sem).wait()


def kernel(x):
    B, D, n = x.shape
    tab = jnp.asarray(_TAB_NP)
    out_t = _sc_kernel(x.reshape(-1), tab)
    return jnp.transpose(out_t, (0, 3, 1, 2))


# speedup vs baseline: 5.6681x; 1.8924x over previous
import functools

import jax
import jax.numpy as jnp
import numpy as np
from jax import lax
from jax.experimental import pallas as pl
from jax.experimental.pallas import tpu as pltpu
from jax.experimental.pallas import tpu_sc as plsc

_POOLING_COUNTS = [15, 8, 8]
_N = 64
_B, _D = 32, 512

_NUM_CORES = 2
_NUM_SUBCORES = 16

_DB = 4
_DBW = 128
_IB = 16
_IBW = 4

_XROW = 0
_P2 = 64
_P4 = 127
_P8 = 188
_P16 = 245
_P32 = 294
_ZROW = 327
_T_ROWS = 328
_PYR = (
    (_P2, _XROW, 1, 63),
    (_P4, _P2, 2, 61),
    (_P8, _P4, 4, 57),
    (_P16, _P8, 8, 49),
    (_P32, _P16, 16, 33),
)
_WROW = {1: _XROW, 2: _P2, 4: _P4, 8: _P8, 16: _P16, 32: _P32}


def _cell_specs():
    maskij = []
    stride, offset = 1, 0
    for c in _POOLING_COUNTS:
        for _ in range(c):
            offset += stride
            i = np.arange(0, _N - offset, stride)
            j = np.arange(offset, _N, stride)
            maskij.append((i, j))
        stride *= 2
    return maskij


def _build_tables():
    cells = []
    for i in range(_N):
        cells.append((i, i, _XROW + i, _XROW + i, _ZROW, _ZROW))
    for (ii, jj) in _cell_specs():
        for i, j in zip(ii.tolist(), jj.tolist()):
            length = j - i + 1
            w = 1
            while w * 2 <= length:
                w *= 2
            w = min(w, 32)
            cells.append((i, j, _WROW[w] + i, _WROW[w] + (j - w + 1),
                          _XROW + i, _XROW + j))
    groups = [[] for _ in range(_IB)]
    for c in cells:
        groups[c[0] // _IBW].append(c)
    rows = []
    cum = [0]
    for ib, g in enumerate(groups):
        if len(g) % 2:
            g = g + [(ib * _IBW + 3, 0, _ZROW, _ZROW, _ZROW, _ZROW)]
        for (i, j, a, bb, xi, xj) in g:
            rows.append((i % _IBW, j, a * _DBW, bb * _DBW,
                         xi * _DBW, xj * _DBW, 0, 0))
        cum.append(cum[-1] + len(g))
    arr = np.array(rows, dtype=np.int32).reshape(-1)
    ch = [c // 2 for c in cum]
    meta = []
    for q in range(_IB // 2):
        row = []
        for par in range(2):
            ib = 2 * q + par
            pib = ib - 2 if ib >= 2 else ib + 14
            row += [ch[ib], ch[ib + 1], ch[pib], ch[pib + 1]]
        row += [0] * 8
        meta.append(row)
    meta = np.array(meta, dtype=np.int32).reshape(-1)
    return np.concatenate([arr, meta]), arr.shape[0] // 8, cum


_TAB_NP, _NCELLS, _CUM = _build_tables()
_QOFF = 8 * _NCELLS

_mesh = plsc.VectorSubcoreMesh(
    core_axis_name="c", subcore_axis_name="s",
    num_cores=_NUM_CORES, num_subcores=_NUM_SUBCORES)


@functools.partial(
    pl.kernel,
    out_type=jax.ShapeDtypeStruct((_B, _N, _N, _D), jnp.float32),
    mesh=_mesh,
    compiler_params=pltpu.CompilerParams(
        needs_layout_passes=False, use_tc_tiling_on_sc=True),
    scratch_types=[
        pltpu.VMEM((_T_ROWS * _DBW,), jnp.float32),
        pltpu.VMEM((_DBW * _N,), jnp.float32),
        pltpu.VMEM((_IBW, _N, _DBW), jnp.float32),
        pltpu.VMEM((_IBW, _N, _DBW), jnp.float32),
        pltpu.VMEM((8 * _NCELLS + 8 * 16,), jnp.int32),
        pltpu.SemaphoreType.DMA,
        pltpu.SemaphoreType.DMA,
    ],
)
def _sc_kernel(x_hbm, tab_hbm, out_hbm, t_v, stage_v, out_v0, out_v1,
               tab_v, sem0, sem1):
    b = lax.axis_index("s") * _NUM_CORES + lax.axis_index("c")
    pltpu.sync_copy(tab_hbm, tab_v)

    z = jnp.zeros((16,), jnp.float32)
    for dd in range(8):
        t_v[pl.ds(_ZROW * _DBW + dd * 16, 16)] = z

    @plsc.parallel_loop(0, _IBW * _N * _DBW // 16, unroll=2)
    def zero_all(m):
        ir = lax.shift_right_logical(m, 9)
        j = lax.bitwise_and(lax.shift_right_logical(m, 3), 63)
        dd16 = lax.bitwise_and(m, 7) * 16
        out_v0[ir, j, pl.ds(dd16, 16)] = z
        out_v1[ir, j, pl.ds(dd16, 16)] = z

    iota64 = lax.iota(jnp.int32, 16) * 64

    def db_body(db, carry):
        pltpu.sync_copy(
            x_hbm.at[pl.ds(b * (_D * _N) + db * (_DBW * _N), _DBW * _N)],
            stage_v)

        @plsc.parallel_loop(0, 8, unroll=2)
        def tr_body(dd):
            base = dd * 1024
            for i in range(_N):
                v = plsc.load_gather(stage_v, [iota64 + (base + i)])
                t_v[pl.ds(i * _DBW + dd * 16, 16)] = v

        @plsc.parallel_loop(0, 8, unroll=2)
        def pyr_body(dd):
            o = dd * 16
            for (pdst, src, shift, cnt) in _PYR:
                for k in range(cnt):
                    va = t_v[pl.ds((src + k) * _DBW + o, 16)]
                    vb = t_v[pl.ds((src + k + shift) * _DBW + o, 16)]
                    t_v[pl.ds((pdst + k) * _DBW + o, 16)] = jnp.maximum(va, vb)

        def q_body(q, c2):
            qm = tab_v[pl.ds(_QOFF + q * 16, 16)]
            for par in range(2):
                buf = out_v0 if par == 0 else out_v1
                sem = sem0 if par == 0 else sem1
                lo = qm[4 * par + 0]
                hi = qm[4 * par + 1]
                rzlo = qm[4 * par + 2]
                rzhi = qm[4 * par + 3]
                dst = out_hbm.at[b, pl.ds(q * 2 * _IBW + par * _IBW, _IBW),
                                 :, pl.ds(db * _DBW, _DBW)]

                @pl.when(jnp.logical_or(db > 0, q > 0))
                def _():
                    pltpu.make_async_copy(buf, dst, sem).wait()

                    @plsc.parallel_loop(rzlo, rzhi, unroll=2)
                    def rezero_body(p):
                        meta = tab_v[pl.ds(p * 16, 16)]
                        for h in range(2):
                            ir = meta[8 * h + 0]
                            j = meta[8 * h + 1]
                            for dd in range(8):
                                buf[ir, j, pl.ds(dd * 16, 16)] = z

                @plsc.parallel_loop(lo, hi, unroll=2)
                def cell_body(p):
                    meta = tab_v[pl.ds(p * 16, 16)]
                    for h in range(2):
                        ir = meta[8 * h + 0]
                        j = meta[8 * h + 1]
                        a = meta[8 * h + 2]
                        bo = meta[8 * h + 3]
                        xi = meta[8 * h + 4]
                        xj = meta[8 * h + 5]
                        for dd in range(8):
                            o = dd * 16
                            va = t_v[pl.ds(a + o, 16)]
                            vb = t_v[pl.ds(bo + o, 16)]
                            vxi = t_v[pl.ds(xi + o, 16)]
                            vxj = t_v[pl.ds(xj + o, 16)]
                            buf[ir, j, pl.ds(o, 16)] = (
                                jnp.maximum(va, vb) + vxi + vxj)

                pltpu.async_copy(buf, dst, sem)
            return c2

        lax.fori_loop(0, _IB // 2, q_body, 0)
        return carry

    lax.fori_loop(0, _DB, db_body, 0)

    for (buf, sem, ib) in ((out_v0, sem0, 14), (out_v1, sem1, 15)):
        pltpu.make_async_copy(
            buf,
            out_hbm.at[b, pl.ds(ib * _IBW, _IBW), :,
                       pl.ds((_DB - 1) * _DBW, _DBW)],
            sem).wait()


def kernel(x):
    B, D, n = x.shape
    tab = jnp.asarray(_TAB_NP)
    out_t = _sc_kernel(x.reshape(-1), tab)
    return jnp.transpose(out_t, (0, 3, 1, 2))


# speedup vs baseline: 5.6771x; 1.0016x over previous
import functools

import jax
import jax.numpy as jnp
import numpy as np
from jax import lax
from jax.experimental import pallas as pl
from jax.experimental.pallas import tpu as pltpu
from jax.experimental.pallas import tpu_sc as plsc

_POOLING_COUNTS = [15, 8, 8]
_N = 64
_B, _D = 32, 512

_NUM_CORES = 2
_NUM_SUBCORES = 16

_DB = 4
_DBW = 128
_IB = 16
_IBW = 4

_XROW = 0
_P2 = 64
_P4 = 127
_P8 = 188
_P16 = 245
_P32 = 294
_ZROW = 327
_T_ROWS = 328
_PYR = (
    (_P2, _XROW, 1, 63),
    (_P4, _P2, 2, 61),
    (_P8, _P4, 4, 57),
    (_P16, _P8, 8, 49),
    (_P32, _P16, 16, 33),
)
_WROW = {1: _XROW, 2: _P2, 4: _P4, 8: _P8, 16: _P16, 32: _P32}


def _cell_specs():
    maskij = []
    stride, offset = 1, 0
    for c in _POOLING_COUNTS:
        for _ in range(c):
            offset += stride
            i = np.arange(0, _N - offset, stride)
            j = np.arange(offset, _N, stride)
            maskij.append((i, j))
        stride *= 2
    return maskij


def _build_tables():
    cells = []
    for i in range(_N):
        cells.append((i, i, _XROW + i, _XROW + i, _ZROW, _ZROW))
    for (ii, jj) in _cell_specs():
        for i, j in zip(ii.tolist(), jj.tolist()):
            length = j - i + 1
            w = 1
            while w * 2 <= length:
                w *= 2
            w = min(w, 32)
            cells.append((i, j, _WROW[w] + i, _WROW[w] + (j - w + 1),
                          _XROW + i, _XROW + j))
    groups = [[] for _ in range(_IB)]
    for c in cells:
        groups[c[0] // _IBW].append(c)
    posset = [{(i % _IBW, j) for (i, j, *_rest) in g} for g in groups]

    order = list(range(_IB - 1, -1, -1))
    rows = []
    cum = [0]
    for p, ib in enumerate(order):
        g = list(groups[ib])
        if len(g) % 2:
            g = g + [(ib * _IBW + 3, 0, _ZROW, _ZROW, _ZROW, _ZROW)]
        if ib + 2 < _IB:
            rz = [(i % _IBW, j) for (i, j, *_r) in groups[ib + 2]
                  if (i % _IBW, j) not in posset[ib]]
        else:
            rz = []
        assert len(rz) <= len(g)
        rz = rz + [(3, 0)] * (len(g) - len(rz))
        for (i, j, a, bb, xi, xj), (rzi, rzj) in zip(g, rz):
            rows.append((i % _IBW, j, a * _DBW, bb * _DBW,
                         xi * _DBW, xj * _DBW, rzi, rzj))
        cum.append(cum[-1] + len(g))
    arr = np.array(rows, dtype=np.int32).reshape(-1)
    ch = [c // 2 for c in cum]
    meta = []
    for q in range(_IB // 2):
        meta.append([ch[2 * q], ch[2 * q + 1],
                     ch[2 * q + 1], ch[2 * q + 2]] + [0] * 12)
    meta = np.array(meta, dtype=np.int32).reshape(-1)
    wrap = []
    wlens = []
    for (old_ib, new_ib) in ((1, _IB - 1), (0, _IB - 2)):
        tgts = [(i % _IBW, j) for (i, j, *_r) in groups[old_ib]
                if (i % _IBW, j) not in posset[new_ib]]
        nrow = (len(tgts) + 7) // 8
        tgts = tgts + [(3, 0)] * (nrow * 8 - len(tgts))
        flat = [v for t in tgts for v in t]
        wrap.append(np.array(flat, dtype=np.int32))
        wlens.append(nrow)
    return (np.concatenate([arr, meta] + wrap), arr.shape[0] // 8, cum,
            wlens)


_TAB_NP, _NCELLS, _CUM, _WLENS = _build_tables()
_QOFF = 8 * _NCELLS
_W0OFF = _QOFF + 8 * 16
_W1OFF = _W0OFF + 16 * _WLENS[0]

_mesh = plsc.VectorSubcoreMesh(
    core_axis_name="c", subcore_axis_name="s",
    num_cores=_NUM_CORES, num_subcores=_NUM_SUBCORES)


@functools.partial(
    pl.kernel,
    out_type=jax.ShapeDtypeStruct((_B, _N, _N, _D), jnp.float32),
    mesh=_mesh,
    compiler_params=pltpu.CompilerParams(
        needs_layout_passes=False, use_tc_tiling_on_sc=True),
    scratch_types=[
        pltpu.VMEM((_T_ROWS * _DBW,), jnp.float32),
        pltpu.VMEM((_DBW * _N,), jnp.float32),
        pltpu.VMEM((_IBW, _N, _DBW), jnp.float32),
        pltpu.VMEM((_IBW, _N, _DBW), jnp.float32),
        pltpu.VMEM((_TAB_NP.shape[0],), jnp.int32),
        pltpu.SemaphoreType.DMA,
        pltpu.SemaphoreType.DMA,
    ],
)
def _sc_kernel(x_hbm, tab_hbm, out_hbm, t_v, stage_v, out_v0, out_v1,
               tab_v, sem0, sem1):
    b = lax.axis_index("s") * _NUM_CORES + lax.axis_index("c")
    pltpu.sync_copy(tab_hbm, tab_v)

    z = jnp.zeros((16,), jnp.float32)
    for dd in range(8):
        t_v[pl.ds(_ZROW * _DBW + dd * 16, 16)] = z

    @plsc.parallel_loop(0, _IBW * _N * _DBW // 16, unroll=2)
    def zero_all(m):
        ir = lax.shift_right_logical(m, 9)
        j = lax.bitwise_and(lax.shift_right_logical(m, 3), 63)
        dd16 = lax.bitwise_and(m, 7) * 16
        out_v0[ir, j, pl.ds(dd16, 16)] = z
        out_v1[ir, j, pl.ds(dd16, 16)] = z

    iota64 = lax.iota(jnp.int32, 16) * 64

    def db_body(db, carry):
        pltpu.sync_copy(
            x_hbm.at[pl.ds(b * (_D * _N) + db * (_DBW * _N), _DBW * _N)],
            stage_v)

        @plsc.parallel_loop(0, 8, unroll=2)
        def tr_body(dd):
            base = dd * 1024
            for i in range(_N):
                v = plsc.load_gather(stage_v, [iota64 + (base + i)])
                t_v[pl.ds(i * _DBW + dd * 16, 16)] = v

        @plsc.parallel_loop(0, 8, unroll=2)
        def pyr_body(dd):
            o = dd * 16
            for (pdst, src, shift, cnt) in _PYR:
                for k in range(cnt):
                    va = t_v[pl.ds((src + k) * _DBW + o, 16)]
                    vb = t_v[pl.ds((src + k + shift) * _DBW + o, 16)]
                    t_v[pl.ds((pdst + k) * _DBW + o, 16)] = jnp.maximum(va, vb)

        def q_body(q, c2):
            qm = tab_v[pl.ds(_QOFF + q * 16, 16)]
            for par in range(2):
                buf = out_v0 if par == 0 else out_v1
                sem = sem0 if par == 0 else sem1
                lo = qm[2 * par + 0]
                hi = qm[2 * par + 1]
                ib_row = (_IB - 1 - 2 * q - par) * _IBW
                dst = out_hbm.at[b, pl.ds(ib_row, _IBW),
                                 :, pl.ds(db * _DBW, _DBW)]

                @pl.when(jnp.logical_or(db > 0, q > 0))
                def _():
                    pltpu.make_async_copy(buf, dst, sem).wait()

                @pl.when(jnp.logical_and(db > 0, q == 0))
                def _():
                    woff = _W0OFF if par == 0 else _W1OFF
                    @plsc.parallel_loop(0, _WLENS[par], unroll=2)
                    def wrap_body(p):
                        meta = tab_v[pl.ds(woff + p * 16, 16)]
                        for h in range(8):
                            ir = meta[2 * h + 0]
                            j = meta[2 * h + 1]
                            for dd in range(8):
                                buf[ir, j, pl.ds(dd * 16, 16)] = z

                @plsc.parallel_loop(lo, hi, unroll=2)
                def cell_body(p):
                    meta = tab_v[pl.ds(p * 16, 16)]
                    for h in range(2):
                        ir = meta[8 * h + 0]
                        j = meta[8 * h + 1]
                        a = meta[8 * h + 2]
                        bo = meta[8 * h + 3]
                        xi = meta[8 * h + 4]
                        xj = meta[8 * h + 5]
                        rzi = meta[8 * h + 6]
                        rzj = meta[8 * h + 7]
                        buf[rzi, rzj, pl.ds(0, 16)] = z
                        for dd in range(8):
                            o = dd * 16
                            va = t_v[pl.ds(a + o, 16)]
                            vb = t_v[pl.ds(bo + o, 16)]
                            vxi = t_v[pl.ds(xi + o, 16)]
                            vxj = t_v[pl.ds(xj + o, 16)]
                            buf[ir, j, pl.ds(o, 16)] = (
                                jnp.maximum(va, vb) + vxi + vxj)
                            if dd > 0:
                                buf[rzi, rzj, pl.ds(o, 16)] = z

                pltpu.async_copy(buf, dst, sem)
            return c2

        lax.fori_loop(0, _IB // 2, q_body, 0)
        return carry

    lax.fori_loop(0, _DB, db_body, 0)

    for (buf, sem, ib) in ((out_v0, sem0, 1), (out_v1, sem1, 0)):
        pltpu.make_async_copy(
            buf,
            out_hbm.at[b, pl.ds(ib * _IBW, _IBW), :,
                       pl.ds((_DB - 1) * _DBW, _DBW)],
            sem).wait()


def kernel(x):
    B, D, n = x.shape
    tab = jnp.asarray(_TAB_NP)
    out_t = _sc_kernel(x.reshape(-1), tab)
    return jnp.transpose(out_t, (0, 3, 1, 2))
